# baseline (speedup 1.0000x reference)
import jax
import jax.numpy as jnp
from jax import lax
from jax.experimental import pallas as pl
from jax.experimental.pallas import tpu as pltpu
from jax.experimental.pallas import tpu_sc as plsc

N_F = 26
VOCAB_SZ = 100000
EMB = 32
BATCH_SZ = 16384

NC, NS, LANES = 2, 16, 16
NW = NC * NS
PLANES = N_F * EMB
PPW = PLANES // NW
CH = 4096
NQ = BATCH_SZ // CH
UNROLL = 16


def _body(xt_hbm, tt_hbm, out_hbm, plane_v, idx_v, out_v, sem_p, sem_o0, sem_o1):
    sem_o = (sem_o0, sem_o1)
    wid = lax.axis_index("s") * NC + lax.axis_index("c")
    p0 = wid * PPW

    def wait_out(p, q):
        b = q % 2
        pltpu.make_async_copy(
            out_v.at[b], out_hbm.at[p, pl.ds(q * CH, CH)], sem_o[b]
        ).wait()

    def do_plane(i, _):
        p = p0 + i
        f = p // EMB
        cp = pltpu.async_copy(tt_hbm.at[p], plane_v, sem_p)

        @pl.when(jnp.logical_or(i == 0, p % EMB == 0))
        def _():
            pltpu.sync_copy(xt_hbm.at[f], idx_v)

        cp.wait()
        for q in range(NQ):
            b = q % 2
            if q < 2:
                @pl.when(i > 0)
                def _():
                    wait_out(p - 1, q + 2)
            else:
                wait_out(p, q - 2)

            @plsc.parallel_loop(0, CH, step=LANES, unroll=UNROLL)
            def _(o):
                ii = idx_v[pl.ds(q * CH + o, LANES)]
                out_v[b, pl.ds(o, LANES)] = plsc.load_gather(plane_v, [ii])
            pltpu.async_copy(out_v.at[b], out_hbm.at[p, pl.ds(q * CH, CH)], sem_o[b])
        return ()

    lax.fori_loop(0, PPW, do_plane, ())
    wait_out(p0 + PPW - 1, NQ - 2)
    wait_out(p0 + PPW - 1, NQ - 1)


@jax.jit
def _embed(xt, tt):
    mesh = plsc.VectorSubcoreMesh(core_axis_name="c", subcore_axis_name="s")
    run = pl.kernel(
        _body,
        out_type=jax.ShapeDtypeStruct((PLANES, BATCH_SZ), jnp.float32),
        mesh=mesh,
        scratch_types=[
            pltpu.VMEM((VOCAB_SZ,), jnp.float32),
            pltpu.VMEM((BATCH_SZ,), jnp.int32),
            pltpu.VMEM((2, CH), jnp.float32),
            pltpu.SemaphoreType.DMA,
            pltpu.SemaphoreType.DMA,
            pltpu.SemaphoreType.DMA,
        ],
        compiler_params=pltpu.CompilerParams(
            use_tc_tiling_on_sc=True, needs_layout_passes=False
        ),
    )
    return run(xt, tt)


def kernel(X, tables):
    xt = X.T
    tt = jnp.transpose(tables, (0, 2, 1)).reshape(PLANES, VOCAB_SZ)
    out_t = _embed(xt, tt)
    return out_t.T.reshape(BATCH_SZ, PLANES)
